# baseline (speedup 1.0000x reference)
import jax
import jax.numpy as jnp
from jax.experimental import pallas as pl

N_NODES = 10000
NUM_GRAPHS = 64
HEADS = 8
CHANNELS = 512
HEAD_DIM = CHANNELS // HEADS
LSTM_DIM = 64


def _mean_pool(data, seg, num_segments):
    s = jax.ops.segment_sum(data, seg, num_segments=num_segments)
    cnt = jax.ops.segment_sum(jnp.ones((data.shape[0],), jnp.float32), seg, num_segments=num_segments)
    return s / jnp.maximum(cnt, 1.0)[:, None]


def _lstm_last(emb, p):
    B = emb.shape[0]
    h0 = jnp.zeros((B, LSTM_DIM), jnp.float32)
    c0 = jnp.zeros((B, LSTM_DIM), jnp.float32)

    def step(carry, xt):
        h, c = carry
        gates = xt @ p['lstm_W_ih'].T + p['lstm_b_ih'] + h @ p['lstm_W_hh'].T + p['lstm_b_hh']
        i, f, g, o = jnp.split(gates, 4, axis=-1)
        c = jax.nn.sigmoid(f) * c + jax.nn.sigmoid(i) * jnp.tanh(g)
        h = jax.nn.sigmoid(o) * jnp.tanh(c)
        return (h, c), None

    (h, _), _ = jax.lax.scan(step, (h0, c0), jnp.transpose(emb, (1, 0, 2)))
    return h


def _segment_softmax(logits, seg, num_segments):
    m = jax.ops.segment_max(logits, seg, num_segments=num_segments)
    m = jnp.where(jnp.isfinite(m), m, 0.0)
    e = jnp.exp(logits - m[seg])
    s = jax.ops.segment_sum(e, seg, num_segments=num_segments)
    return e / (s[seg] + 1e-16)


def _gatv2(h, src, dst, edge_attr, p, num_nodes):
    xl = (h @ p['Wl'] + p['bl'])[src].reshape(-1, HEADS, HEAD_DIM)
    xr = (h @ p['Wr'] + p['br'])[dst].reshape(-1, HEADS, HEAD_DIM)
    ee = (edge_attr @ p['We']).reshape(-1, HEADS, HEAD_DIM)
    m = jax.nn.leaky_relu(xl + xr + ee, negative_slope=0.2)
    alpha = jnp.einsum('ehc,hc->eh', m, p['att'])
    alpha = _segment_softmax(alpha, dst, num_nodes)
    out = jax.ops.segment_sum(xl * alpha[:, :, None], dst, num_segments=num_nodes)
    return out.reshape(num_nodes, CHANNELS) + p['bias']


def _touch(x):
    def body(x_ref, o_ref):
        o_ref[...] = x_ref[...]
    return pl.pallas_call(body, out_shape=jax.ShapeDtypeStruct(x.shape, x.dtype))(x)


def kernel(x, edge_index, edge_attr, batch, params):
    p = params
    states = x[:, 0]
    addl = x[:, 1:].astype(jnp.float32)
    node_x = jnp.concatenate([p['state_number_embed'][states], addl], axis=-1)
    src_sn = edge_attr[:, -2]
    tgt_sn = edge_attr[:, -1]
    regex = edge_attr[:, :-2]
    src_emb = p['state_number_embed'][src_sn]
    tgt_emb = p['state_number_embed'][tgt_sn]
    enc = _lstm_last(p['regex_embed'][regex], p)
    src = edge_index[0]
    dst = edge_index[1]
    out_trans = _mean_pool(jnp.concatenate([tgt_emb, enc], axis=-1), src, N_NODES)
    in_trans = _mean_pool(jnp.concatenate([src_emb, enc], axis=-1), dst, N_NODES)
    h = jnp.concatenate([node_x, in_trans, out_trans], axis=-1)
    h = jax.nn.relu(_gatv2(h, src, dst, enc, p['conv1'], N_NODES))
    h = jax.nn.relu(_gatv2(h, src, dst, enc, p['conv2'], N_NODES)) + h
    h = jax.nn.relu(_gatv2(h, src, dst, enc, p['conv3'], N_NODES)) + h
    h = jax.nn.relu(_gatv2(h, src, dst, enc, p['conv4'], N_NODES)) + h
    s = _mean_pool(h, batch, NUM_GRAPHS)
    s = _touch(s)
    pi = s @ p['policy_head2_W'].T + p['policy_head2_b']
    v = jax.nn.relu(s @ p['value_head1_W'].T + p['value_head1_b'])
    v = v @ p['value_head2_W'].T + p['value_head2_b']
    return jax.nn.log_softmax(pi, axis=1), v


# speedup vs baseline: 5.8078x; 5.8078x over previous
import functools

import jax
import jax.numpy as jnp
from jax import lax
from jax.experimental import pallas as pl
from jax.experimental.pallas import tpu as pltpu
from jax.experimental.pallas import tpu_sc as plsc

F32 = jnp.float32
I32 = jnp.int32

NN = 10000
NE = 160000
NP = 10240
EP = 163840
NG = 64
HEADS = 8
CH = 512
HD = 64
LD = 64
NW = 32


def _sc_mesh():
    return plsc.VectorSubcoreMesh(core_axis_name="c", subcore_axis_name="s",
                                  num_cores=2, num_subcores=16)


@functools.lru_cache(None)
def _make_gather(T, D, NSB):
    nper = -(-NSB // NW)

    @functools.partial(
        pl.kernel,
        out_type=jax.ShapeDtypeStruct((NSB * 128, D), F32),
        mesh=_sc_mesh(),
        scratch_types=[
            pltpu.VMEM((128,), I32),
            pltpu.VMEM((128, D), F32),
            pltpu.SemaphoreType.DMA,
        ],
    )
    def k(table_hbm, idx_hbm, out_hbm, idx_v, rows_v, sem):
        wid = lax.axis_index("s") * 2 + lax.axis_index("c")

        def body(i, carry):
            sb = wid + i * NW

            @pl.when(sb < NSB)
            def _():
                pltpu.sync_copy(idx_hbm.at[sb], idx_v)
                pltpu.async_copy(table_hbm.at[idx_v], rows_v, sem).wait()
                pltpu.sync_copy(rows_v, out_hbm.at[pl.ds(sb * 128, 128)])

            return carry

        lax.fori_loop(0, nper, body, 0)

    return k


def _gather(table, idx2):
    T, D = table.shape
    NSB = idx2.shape[0]
    return _make_gather(T, D, NSB)(table, idx2)


@functools.lru_cache(None)
def _make_scatter(N, Dc, NCH, NSB):
    nper = -(-NSB // NW)
    ZT = min(16, N // 8)
    rpt = N // ZT

    @functools.partial(
        pl.kernel,
        out_type=jax.ShapeDtypeStruct((NCH, 2, N, Dc), F32),
        mesh=_sc_mesh(),
        scratch_types=[
            pltpu.VMEM((128,), I32),
            pltpu.VMEM((128, Dc), F32),
            pltpu.VMEM_SHARED((N, Dc), F32),
            pltpu.SemaphoreType.DMA,
        ],
    )
    def k(vals_hbm, idx_hbm, zeros_hbm, out_hbm, idx_v, val_v, acc, sem):
        cc = lax.axis_index("c")
        ss = lax.axis_index("s")
        wid = ss * 2 + cc
        for ch in range(NCH):
            @pl.when(ss < ZT)
            def _():
                pltpu.sync_copy(zeros_hbm.at[pl.ds(ss * rpt, rpt)],
                                acc.at[pl.ds(ss * rpt, rpt)])
            plsc.subcore_barrier()

            def body(i, carry):
                sb = wid + i * NW

                @pl.when(sb < NSB)
                def _():
                    pltpu.sync_copy(idx_hbm.at[sb], idx_v)
                    pltpu.sync_copy(vals_hbm.at[ch, pl.ds(sb * 128, 128)], val_v)
                    pltpu.sync_copy(val_v, acc.at[idx_v], add=True)

                return carry

            lax.fori_loop(0, nper, body, 0)
            plsc.subcore_barrier()

            @pl.when(ss < ZT)
            def _():
                pltpu.sync_copy(acc.at[pl.ds(ss * rpt, rpt)],
                                out_hbm.at[ch, cc, pl.ds(ss * rpt, rpt)])

    return k


def _scatter(values, idx2, N, zeros):
    NCH, ROWS, Dc = values.shape
    NSB = idx2.shape[0]
    return _make_scatter(N, Dc, NCH, NSB)(values, idx2, zeros)


def _xlxr(X, Wl, bl, Wr, br):
    M, K = X.shape
    N = Wl.shape[1]
    BM = 1024

    def body(x_ref, wl_ref, bl_ref, wr_ref, br_ref, o1_ref, o2_ref):
        xv = x_ref[...]
        o1_ref[...] = jnp.dot(xv, wl_ref[...], preferred_element_type=F32) + bl_ref[...]
        o2_ref[...] = jnp.dot(xv, wr_ref[...], preferred_element_type=F32) + br_ref[...]

    return pl.pallas_call(
        body,
        grid=(M // BM,),
        in_specs=[
            pl.BlockSpec((BM, K), lambda i: (i, 0)),
            pl.BlockSpec((K, N), lambda i: (0, 0)),
            pl.BlockSpec((1, N), lambda i: (0, 0)),
            pl.BlockSpec((K, N), lambda i: (0, 0)),
            pl.BlockSpec((1, N), lambda i: (0, 0)),
        ],
        out_specs=[
            pl.BlockSpec((BM, N), lambda i: (i, 0)),
            pl.BlockSpec((BM, N), lambda i: (i, 0)),
        ],
        out_shape=[
            jax.ShapeDtypeStruct((M, N), F32),
            jax.ShapeDtypeStruct((M, N), F32),
        ],
    )(X, Wl, bl, Wr, br)


def _lstm(xg, tgt_e, src_e, wih_t, whh_t, b):
    BE = 2048

    def body(xg_ref, te_ref, se_ref, wih_ref, whh_ref, b_ref, enc_ref, ps_ref, pd_ref):
        xv = xg_ref[...][:, :, 0:32]
        wih = wih_ref[...]
        whh = whh_ref[...]
        bb = b_ref[...]
        h = jnp.zeros((BE, LD), F32)
        c = jnp.zeros((BE, LD), F32)
        for t in range(8):
            g = (jnp.dot(xv[:, t, :], wih, preferred_element_type=F32)
                 + jnp.dot(h, whh, preferred_element_type=F32) + bb)
            gi = jax.nn.sigmoid(g[:, 0:LD])
            gf = jax.nn.sigmoid(g[:, LD:2 * LD])
            gg = jnp.tanh(g[:, 2 * LD:3 * LD])
            go = jax.nn.sigmoid(g[:, 3 * LD:4 * LD])
            c = gf * c + gi * gg
            h = go * jnp.tanh(c)
        row = pl.program_id(0) * BE + lax.broadcasted_iota(I32, (BE, 1), 0)
        mask = row < NE
        enc_ref[...] = jnp.where(mask, h, 0.0)
        ones = jnp.ones((BE, 1), F32)
        zpad = jnp.zeros((BE, 31), F32)
        pvs = jnp.concatenate([te_ref[...][:, 0:32], h, ones, zpad], axis=1)
        pvd = jnp.concatenate([se_ref[...][:, 0:32], h, ones, zpad], axis=1)
        ps_ref[...] = jnp.where(mask, pvs, 0.0)
        pd_ref[...] = jnp.where(mask, pvd, 0.0)

    return pl.pallas_call(
        body,
        grid=(EP // BE,),
        in_specs=[
            pl.BlockSpec((BE, 8, 128), lambda i: (i, 0, 0)),
            pl.BlockSpec((BE, 128), lambda i: (i, 0)),
            pl.BlockSpec((BE, 128), lambda i: (i, 0)),
            pl.BlockSpec((32, 256), lambda i: (0, 0)),
            pl.BlockSpec((64, 256), lambda i: (0, 0)),
            pl.BlockSpec((1, 256), lambda i: (0, 0)),
        ],
        out_specs=[
            pl.BlockSpec((BE, LD), lambda i: (i, 0)),
            pl.BlockSpec((BE, 128), lambda i: (i, 0)),
            pl.BlockSpec((BE, 128), lambda i: (i, 0)),
        ],
        out_shape=[
            jax.ShapeDtypeStruct((EP, LD), F32),
            jax.ShapeDtypeStruct((EP, 128), F32),
            jax.ShapeDtypeStruct((EP, 128), F32),
        ],
    )(xg, tgt_e, src_e, wih_t, whh_t, b)


def _hbuild(emb_s, addl, as0, as1, ad0, ad1):
    BM = 2048

    def body(es_ref, adl_ref, as0_ref, as1_ref, ad0_ref, ad1_ref, o_ref):
        es = es_ref[...][:, 0:32]
        a_s = as0_ref[...] + as1_ref[...]
        a_d = ad0_ref[...] + ad1_ref[...]
        cs = jnp.maximum(a_s[:, 96:97], 1.0)
        cd = jnp.maximum(a_d[:, 96:97], 1.0)
        out_trans = a_s[:, 0:96] / cs
        in_trans = a_d[:, 0:96] / cd
        zpad = jnp.zeros((BM, 30), F32)
        o_ref[...] = jnp.concatenate(
            [es, adl_ref[...][:, 0:2], in_trans, out_trans, zpad], axis=1)

    return pl.pallas_call(
        body,
        grid=(NP // BM,),
        in_specs=[pl.BlockSpec((BM, 128), lambda i: (i, 0))] * 6,
        out_specs=pl.BlockSpec((BM, 256), lambda i: (i, 0)),
        out_shape=jax.ShapeDtypeStruct((NP, 256), F32),
    )(emb_s, addl, as0, as1, ad0, ad1)


def _ee_logits(xls, xrd, enc, we, attf):
    BE = 2048

    def body(xl_ref, xr_ref, en_ref, we_ref, at_ref, l_ref):
        ee = jnp.dot(en_ref[...], we_ref[...], preferred_element_type=F32)
        t = xl_ref[...] + xr_ref[...] + ee
        t = jnp.where(t > 0, t, 0.2 * t)
        tw = t * at_ref[...]
        L = jnp.sum(tw.reshape(BE, HEADS, HD), axis=2)
        row = pl.program_id(0) * BE + lax.broadcasted_iota(I32, (BE, 1), 0)
        mask = row < NE
        one = jnp.where(mask, jnp.ones((BE, 1), F32), 0.0)
        Lm = jnp.where(mask, L, 0.0)
        l_ref[...] = jnp.concatenate([Lm, one, jnp.zeros((BE, 7), F32)], axis=1)

    return pl.pallas_call(
        body,
        grid=(EP // BE,),
        in_specs=[
            pl.BlockSpec((BE, CH), lambda i: (i, 0)),
            pl.BlockSpec((BE, CH), lambda i: (i, 0)),
            pl.BlockSpec((BE, LD), lambda i: (i, 0)),
            pl.BlockSpec((LD, CH), lambda i: (0, 0)),
            pl.BlockSpec((1, CH), lambda i: (0, 0)),
        ],
        out_specs=pl.BlockSpec((BE, 16), lambda i: (i, 0)),
        out_shape=jax.ShapeDtypeStruct((EP, 16), F32),
    )(xls, xrd, enc, we, attf)


def _segmean(al0, al1):
    BM = 2048

    def body(a0_ref, a1_ref, o_ref):
        a = a0_ref[...] + a1_ref[...]
        cnt = jnp.maximum(a[:, 8:9], 1.0)
        o_ref[...] = jnp.concatenate(
            [a[:, 0:8] / cnt, jnp.zeros((BM, 120), F32)], axis=1)

    return pl.pallas_call(
        body,
        grid=(NP // BM,),
        in_specs=[pl.BlockSpec((BM, 16), lambda i: (i, 0))] * 2,
        out_specs=pl.BlockSpec((BM, 128), lambda i: (i, 0)),
        out_shape=jax.ShapeDtypeStruct((NP, 128), F32),
    )(al0, al1)


def _weights(L16, med, xls):
    BE = 2048

    def body(l_ref, m_ref, xl_ref, wc_ref, e_ref):
        lv = l_ref[...]
        L = lv[:, 0:8]
        live = lv[:, 8:9] > 0
        E = jnp.exp(jnp.minimum(L - m_ref[...][:, 0:8], 60.0))
        E = jnp.where(live, E, 0.0)
        e_ref[...] = jnp.concatenate([E, jnp.zeros((BE, 8), F32)], axis=1)
        xv = xl_ref[...]
        parts = []
        for ch in range(4):
            e2 = E[:, 2 * ch:2 * ch + 2]
            er = jnp.broadcast_to(e2[:, :, None], (BE, 2, HD)).reshape(BE, 128)
            parts.append((xv[:, ch * 128:(ch + 1) * 128] * er)[None])
        wc_ref[...] = jnp.concatenate(parts, axis=0)

    return pl.pallas_call(
        body,
        grid=(EP // BE,),
        in_specs=[
            pl.BlockSpec((BE, 16), lambda i: (i, 0)),
            pl.BlockSpec((BE, 128), lambda i: (i, 0)),
            pl.BlockSpec((BE, CH), lambda i: (i, 0)),
        ],
        out_specs=[
            pl.BlockSpec((4, BE, 128), lambda i: (0, i, 0)),
            pl.BlockSpec((BE, 16), lambda i: (i, 0)),
        ],
        out_shape=[
            jax.ShapeDtypeStruct((4, EP, 128), F32),
            jax.ShapeDtypeStruct((EP, 16), F32),
        ],
    )(L16, med, xls)


@functools.lru_cache(None)
def _make_finalize(residual):
    BM = 2048

    def body(*refs):
        ws = refs[0:8]
        e0_ref, e1_ref, b_ref = refs[8:11]
        if residual:
            hp_ref, h_ref, hc_ref = refs[11:14]
        else:
            h_ref, hc_ref = refs[11:13]
        s = (e0_ref[...] + e1_ref[...])[:, 0:HEADS]
        parts = []
        for ch in range(4):
            a = ws[2 * ch][...] + ws[2 * ch + 1][...]
            s2 = s[:, 2 * ch:2 * ch + 2]
            den = jnp.broadcast_to(s2[:, :, None], (BM, 2, HD)).reshape(BM, 128)
            den = jnp.where(den > 0, den, 1.0)
            parts.append(a / den + b_ref[...][:, ch * 128:(ch + 1) * 128])
        h = jax.nn.relu(jnp.concatenate(parts, axis=1))
        if residual:
            h = h + hp_ref[...]
        row = pl.program_id(0) * BM + lax.broadcasted_iota(I32, (BM, 1), 0)
        h = jnp.where(row < NN, h, 0.0)
        h_ref[...] = h
        hc_ref[...] = jnp.concatenate(
            [h[:, ch * 128:(ch + 1) * 128][None] for ch in range(4)], axis=0)

    in_specs = ([pl.BlockSpec((BM, 128), lambda i: (i, 0))] * 8
                + [pl.BlockSpec((BM, 16), lambda i: (i, 0))] * 2
                + [pl.BlockSpec((1, CH), lambda i: (0, 0))])
    if residual:
        in_specs.append(pl.BlockSpec((BM, CH), lambda i: (i, 0)))

    return pl.pallas_call(
        body,
        grid=(NP // BM,),
        in_specs=in_specs,
        out_specs=[
            pl.BlockSpec((BM, CH), lambda i: (i, 0)),
            pl.BlockSpec((4, BM, 128), lambda i: (0, i, 0)),
        ],
        out_shape=[
            jax.ShapeDtypeStruct((NP, CH), F32),
            jax.ShapeDtypeStruct((4, NP, 128), F32),
        ],
    )


def _head(abs_, cnts, p2t, pb, v1t, v1b, v2t, v2b):
    def body(a0, a1, a2, a3, a4, a5, a6, a7, c0, c1, p2_ref, pb_ref,
             v1_ref, vb1_ref, v2_ref, vb2_ref, pi_ref, v_ref):
        cnt = jnp.maximum((c0[...] + c1[...])[:, 0:1], 1.0)
        aa = (a0, a1, a2, a3, a4, a5, a6, a7)
        parts = [(aa[2 * ch][...] + aa[2 * ch + 1][...]) / cnt for ch in range(4)]
        s = jnp.concatenate(parts, axis=1)
        pi = jnp.dot(s, p2_ref[...], preferred_element_type=F32) + pb_ref[...]
        m = jnp.max(pi, axis=1, keepdims=True)
        lse = jnp.log(jnp.sum(jnp.exp(pi - m), axis=1, keepdims=True)) + m
        pi_ref[...] = pi - lse
        v = jax.nn.relu(jnp.dot(s, v1_ref[...], preferred_element_type=F32) + vb1_ref[...])
        v_ref[...] = jnp.dot(v, v2_ref[...], preferred_element_type=F32) + vb2_ref[...]

    return pl.pallas_call(
        body,
        out_shape=[
            jax.ShapeDtypeStruct((NG, CH), F32),
            jax.ShapeDtypeStruct((NG, 128), F32),
        ],
    )(*abs_, *cnts, p2t, pb, v1t, v1b, v2t, v2b)


def _pad_idx(a, n):
    return jnp.pad(a.astype(I32), (0, n - a.shape[0])).reshape(-1, 128)


def kernel(x, edge_index, edge_attr, batch, params):
    p = params
    emb = jnp.pad(p['state_number_embed'], ((0, 0), (0, 96)))
    remb = jnp.pad(p['regex_embed'], ((0, 0), (0, 96)))

    states2 = _pad_idx(x[:, 0], NP)
    src2 = _pad_idx(edge_index[0], EP)
    dst2 = _pad_idx(edge_index[1], EP)
    src_sn2 = _pad_idx(edge_attr[:, -2], EP)
    tgt_sn2 = _pad_idx(edge_attr[:, -1], EP)
    regex2 = _pad_idx(edge_attr[:, :8].reshape(-1), EP * 8)
    batch2 = _pad_idx(batch, NP)
    addl_pad = jnp.pad(x[:, 1:].astype(F32), ((0, NP - NN), (0, 126)))
    ones16 = jnp.pad((jnp.arange(NP) < NN).astype(F32)[:, None], ((0, 0), (0, 15)))
    zeros_n128 = jnp.zeros((NP, 128), F32)
    zeros_n16 = jnp.zeros((NP, 16), F32)
    zeros_g128 = jnp.zeros((NG, 128), F32)
    zeros_g16 = jnp.zeros((NG, 16), F32)

    wih_t = p['lstm_W_ih'].T
    whh_t = p['lstm_W_hh'].T
    lb = (p['lstm_b_ih'] + p['lstm_b_hh'])[None]

    emb_s = _gather(emb, states2)
    src_e = _gather(emb, src_sn2)
    tgt_e = _gather(emb, tgt_sn2)
    xg = _gather(remb, regex2).reshape(EP, 8, 128)

    enc, pv_src, pv_dst = _lstm(xg, tgt_e, src_e, wih_t, whh_t, lb)

    acc_s = _scatter(pv_src[None], src2, NP, zeros_n128)
    acc_d = _scatter(pv_dst[None], dst2, NP, zeros_n128)

    h = _hbuild(emb_s, addl_pad, acc_s[0, 0], acc_s[0, 1],
                acc_d[0, 0], acc_d[0, 1])

    hc = None
    for li, name in enumerate(['conv1', 'conv2', 'conv3', 'conv4']):
        cp = p[name]
        K = h.shape[1]
        wl = jnp.pad(cp['Wl'], ((0, K - cp['Wl'].shape[0]), (0, 0)))
        wr = jnp.pad(cp['Wr'], ((0, K - cp['Wr'].shape[0]), (0, 0)))
        xl, xr = _xlxr(h, wl, cp['bl'][None], wr, cp['br'][None])
        xls = _gather(xl, src2)
        xrd = _gather(xr, dst2)
        attf = cp['att'].reshape(1, CH)
        L16 = _ee_logits(xls, xrd, enc, cp['We'], attf)
        accl = _scatter(L16[None], dst2, NP, zeros_n16)
        med = _segmean(accl[0, 0], accl[0, 1])
        mede = _gather(med, dst2)
        wc, e16 = _weights(L16, mede, xls)
        acc_w = _scatter(wc, dst2, NP, zeros_n128)
        acc_e = _scatter(e16[None], dst2, NP, zeros_n16)
        fin = _make_finalize(li > 0)
        args = tuple(acc_w[ch, c] for ch in range(4) for c in range(2)) + (
            acc_e[0, 0], acc_e[0, 1], cp['bias'][None])
        if li > 0:
            args = args + (h,)
        h, hc = fin(*args)

    acc_b = _scatter(hc, batch2, NG, zeros_g128)
    cnt_b = _scatter(ones16[None], batch2, NG, zeros_g16)

    p2t = p['policy_head2_W'].T
    v1t = p['value_head1_W'].T
    v2t = jnp.pad(p['value_head2_W'].T, ((0, 0), (0, 127)))
    v2b = jnp.pad(p['value_head2_b'][None], ((0, 0), (0, 127)))
    abs_ = tuple(acc_b[ch, c] for ch in range(4) for c in range(2))
    cnts = (cnt_b[0, 0], cnt_b[0, 1])
    logpi, vout = _head(abs_, cnts, p2t, p['policy_head2_b'][None],
                        v1t, p['value_head1_b'][None], v2t, v2b)
    return logpi, vout[:, :1]


# speedup vs baseline: 6.1915x; 1.0661x over previous
import functools

import jax
import jax.numpy as jnp
from jax import lax
from jax.experimental import pallas as pl
from jax.experimental.pallas import tpu as pltpu
from jax.experimental.pallas import tpu_sc as plsc

F32 = jnp.float32
I32 = jnp.int32

NN = 10000
NE = 160000
NP = 10240
EP = 163840
NG = 64
HEADS = 8
CH = 512
HD = 64
LD = 64
NW = 32


def _sc_mesh():
    return plsc.VectorSubcoreMesh(core_axis_name="c", subcore_axis_name="s",
                                  num_cores=2, num_subcores=16)


@functools.lru_cache(None)
def _make_gather(T, D, NSB):
    nper = -(-NSB // NW)

    @functools.partial(
        pl.kernel,
        out_type=jax.ShapeDtypeStruct((NSB * 128, D), F32),
        mesh=_sc_mesh(),
        scratch_types=[
            pltpu.VMEM((128,), I32),
            pltpu.VMEM((128, D), F32),
            pltpu.SemaphoreType.DMA,
        ],
    )
    def k(table_hbm, idx_hbm, out_hbm, idx_v, rows_v, sem):
        wid = lax.axis_index("s") * 2 + lax.axis_index("c")

        def body(i, carry):
            sb = wid + i * NW

            @pl.when(sb < NSB)
            def _():
                pltpu.sync_copy(idx_hbm.at[sb], idx_v)
                pltpu.async_copy(table_hbm.at[idx_v], rows_v, sem).wait()
                pltpu.sync_copy(rows_v, out_hbm.at[pl.ds(sb * 128, 128)])

            return carry

        lax.fori_loop(0, nper, body, 0)

    return k


def _gather(table, idx2):
    T, D = table.shape
    NSB = idx2.shape[0]
    return _make_gather(T, D, NSB)(table, idx2)


@functools.lru_cache(None)
def _make_scatter(N, Dc, NCH, NSB):
    nper = -(-NSB // NW)
    ZT = min(16, N // 8)
    rpt = N // ZT

    pipelined = NSB % NW == 0

    @functools.partial(
        pl.kernel,
        out_type=jax.ShapeDtypeStruct((NCH, 2, N, Dc), F32),
        mesh=_sc_mesh(),
        scratch_types=[
            pltpu.VMEM((128,), I32),
            pltpu.VMEM((128, Dc), F32),
            pltpu.VMEM((128, Dc), F32),
            pltpu.VMEM_SHARED((N, Dc), F32),
            pltpu.SemaphoreType.DMA,
            pltpu.SemaphoreType.DMA,
        ],
    )
    def k(vals_hbm, idx_hbm, zeros_hbm, out_hbm, idx_v, val_a, val_b, acc,
          sem_a, sem_b):
        cc = lax.axis_index("c")
        ss = lax.axis_index("s")
        wid = ss * 2 + cc
        bufs = [(val_a, sem_a), (val_b, sem_b)]
        for ch in range(NCH):
            @pl.when(ss < ZT)
            def _():
                pltpu.sync_copy(zeros_hbm.at[pl.ds(ss * rpt, rpt)],
                                acc.at[pl.ds(ss * rpt, rpt)])
            plsc.subcore_barrier()

            if pipelined:
                buf0, s0 = bufs[0]
                cur = pltpu.async_copy(
                    vals_hbm.at[ch, pl.ds(wid * 128, 128)], buf0, s0)
                for i in range(nper):
                    nxt = None
                    if i + 1 < nper:
                        bufn, sn = bufs[(i + 1) % 2]
                        nxt = pltpu.async_copy(
                            vals_hbm.at[ch, pl.ds((wid + (i + 1) * NW) * 128, 128)],
                            bufn, sn)
                    cur.wait()
                    pltpu.sync_copy(idx_hbm.at[wid + i * NW], idx_v)
                    pltpu.sync_copy(bufs[i % 2][0], acc.at[idx_v], add=True)
                    cur = nxt
            else:
                def body(i, carry):
                    sb = wid + i * NW

                    @pl.when(sb < NSB)
                    def _():
                        pltpu.sync_copy(idx_hbm.at[sb], idx_v)
                        pltpu.sync_copy(vals_hbm.at[ch, pl.ds(sb * 128, 128)],
                                        val_a)
                        pltpu.sync_copy(val_a, acc.at[idx_v], add=True)

                    return carry

                lax.fori_loop(0, nper, body, 0)
            plsc.subcore_barrier()

            @pl.when(ss < ZT)
            def _():
                pltpu.sync_copy(acc.at[pl.ds(ss * rpt, rpt)],
                                out_hbm.at[ch, cc, pl.ds(ss * rpt, rpt)])

    return k


def _scatter(values, idx2, N, zeros):
    NCH, ROWS, Dc = values.shape
    NSB = idx2.shape[0]
    return _make_scatter(N, Dc, NCH, NSB)(values, idx2, zeros)


def _xlxr(X, Wl, bl, Wr, br):
    M, K = X.shape
    N = Wl.shape[1]
    BM = 1024

    def body(x_ref, wl_ref, bl_ref, wr_ref, br_ref, o1_ref, o2_ref):
        xv = x_ref[...]
        o1_ref[...] = jnp.dot(xv, wl_ref[...], preferred_element_type=F32) + bl_ref[...]
        o2_ref[...] = jnp.dot(xv, wr_ref[...], preferred_element_type=F32) + br_ref[...]

    return pl.pallas_call(
        body,
        grid=(M // BM,),
        in_specs=[
            pl.BlockSpec((BM, K), lambda i: (i, 0)),
            pl.BlockSpec((K, N), lambda i: (0, 0)),
            pl.BlockSpec((1, N), lambda i: (0, 0)),
            pl.BlockSpec((K, N), lambda i: (0, 0)),
            pl.BlockSpec((1, N), lambda i: (0, 0)),
        ],
        out_specs=[
            pl.BlockSpec((BM, N), lambda i: (i, 0)),
            pl.BlockSpec((BM, N), lambda i: (i, 0)),
        ],
        out_shape=[
            jax.ShapeDtypeStruct((M, N), F32),
            jax.ShapeDtypeStruct((M, N), F32),
        ],
    )(X, Wl, bl, Wr, br)


def _lstm(xg, tgt_e, src_e, wih_t, whh_t, b):
    BE = 2048

    def body(xg_ref, te_ref, se_ref, wih_ref, whh_ref, b_ref, enc_ref, ps_ref, pd_ref):
        xv = xg_ref[...][:, :, 0:32]
        wih = wih_ref[...]
        whh = whh_ref[...]
        bb = b_ref[...]
        h = jnp.zeros((BE, LD), F32)
        c = jnp.zeros((BE, LD), F32)
        for t in range(8):
            g = (jnp.dot(xv[:, t, :], wih, preferred_element_type=F32)
                 + jnp.dot(h, whh, preferred_element_type=F32) + bb)
            gi = jax.nn.sigmoid(g[:, 0:LD])
            gf = jax.nn.sigmoid(g[:, LD:2 * LD])
            gg = jnp.tanh(g[:, 2 * LD:3 * LD])
            go = jax.nn.sigmoid(g[:, 3 * LD:4 * LD])
            c = gf * c + gi * gg
            h = go * jnp.tanh(c)
        row = pl.program_id(0) * BE + lax.broadcasted_iota(I32, (BE, 1), 0)
        mask = row < NE
        enc_ref[...] = jnp.where(mask, h, 0.0)
        ones = jnp.ones((BE, 1), F32)
        zpad = jnp.zeros((BE, 31), F32)
        pvs = jnp.concatenate([te_ref[...][:, 0:32], h, ones, zpad], axis=1)
        pvd = jnp.concatenate([se_ref[...][:, 0:32], h, ones, zpad], axis=1)
        ps_ref[...] = jnp.where(mask, pvs, 0.0)
        pd_ref[...] = jnp.where(mask, pvd, 0.0)

    return pl.pallas_call(
        body,
        grid=(EP // BE,),
        in_specs=[
            pl.BlockSpec((BE, 8, 128), lambda i: (i, 0, 0)),
            pl.BlockSpec((BE, 128), lambda i: (i, 0)),
            pl.BlockSpec((BE, 128), lambda i: (i, 0)),
            pl.BlockSpec((32, 256), lambda i: (0, 0)),
            pl.BlockSpec((64, 256), lambda i: (0, 0)),
            pl.BlockSpec((1, 256), lambda i: (0, 0)),
        ],
        out_specs=[
            pl.BlockSpec((BE, LD), lambda i: (i, 0)),
            pl.BlockSpec((BE, 128), lambda i: (i, 0)),
            pl.BlockSpec((BE, 128), lambda i: (i, 0)),
        ],
        out_shape=[
            jax.ShapeDtypeStruct((EP, LD), F32),
            jax.ShapeDtypeStruct((EP, 128), F32),
            jax.ShapeDtypeStruct((EP, 128), F32),
        ],
    )(xg, tgt_e, src_e, wih_t, whh_t, b)


def _hbuild(emb_s, addl, as0, as1, ad0, ad1):
    BM = 2048

    def body(es_ref, adl_ref, as0_ref, as1_ref, ad0_ref, ad1_ref, o_ref):
        es = es_ref[...][:, 0:32]
        a_s = as0_ref[...] + as1_ref[...]
        a_d = ad0_ref[...] + ad1_ref[...]
        cs = jnp.maximum(a_s[:, 96:97], 1.0)
        cd = jnp.maximum(a_d[:, 96:97], 1.0)
        out_trans = a_s[:, 0:96] / cs
        in_trans = a_d[:, 0:96] / cd
        zpad = jnp.zeros((BM, 30), F32)
        o_ref[...] = jnp.concatenate(
            [es, adl_ref[...][:, 0:2], in_trans, out_trans, zpad], axis=1)

    return pl.pallas_call(
        body,
        grid=(NP // BM,),
        in_specs=[pl.BlockSpec((BM, 128), lambda i: (i, 0))] * 6,
        out_specs=pl.BlockSpec((BM, 256), lambda i: (i, 0)),
        out_shape=jax.ShapeDtypeStruct((NP, 256), F32),
    )(emb_s, addl, as0, as1, ad0, ad1)


def _ee_logits(xls, xrd, enc, we, attf):
    BE = 2048

    def body(xl_ref, xr_ref, en_ref, we_ref, at_ref, l_ref):
        ee = jnp.dot(en_ref[...], we_ref[...], preferred_element_type=F32)
        t = xl_ref[...] + xr_ref[...] + ee
        t = jnp.where(t > 0, t, 0.2 * t)
        tw = t * at_ref[...]
        L = jnp.sum(tw.reshape(BE, HEADS, HD), axis=2)
        row = pl.program_id(0) * BE + lax.broadcasted_iota(I32, (BE, 1), 0)
        mask = row < NE
        one = jnp.where(mask, jnp.ones((BE, 1), F32), 0.0)
        Lm = jnp.where(mask, L, 0.0)
        l_ref[...] = jnp.concatenate([Lm, one, jnp.zeros((BE, 7), F32)], axis=1)

    return pl.pallas_call(
        body,
        grid=(EP // BE,),
        in_specs=[
            pl.BlockSpec((BE, CH), lambda i: (i, 0)),
            pl.BlockSpec((BE, CH), lambda i: (i, 0)),
            pl.BlockSpec((BE, LD), lambda i: (i, 0)),
            pl.BlockSpec((LD, CH), lambda i: (0, 0)),
            pl.BlockSpec((1, CH), lambda i: (0, 0)),
        ],
        out_specs=pl.BlockSpec((BE, 16), lambda i: (i, 0)),
        out_shape=jax.ShapeDtypeStruct((EP, 16), F32),
    )(xls, xrd, enc, we, attf)


def _segmean(al0, al1):
    BM = 2048

    def body(a0_ref, a1_ref, o_ref):
        a = a0_ref[...] + a1_ref[...]
        cnt = jnp.maximum(a[:, 8:9], 1.0)
        o_ref[...] = jnp.concatenate(
            [a[:, 0:8] / cnt, jnp.zeros((BM, 120), F32)], axis=1)

    return pl.pallas_call(
        body,
        grid=(NP // BM,),
        in_specs=[pl.BlockSpec((BM, 16), lambda i: (i, 0))] * 2,
        out_specs=pl.BlockSpec((BM, 128), lambda i: (i, 0)),
        out_shape=jax.ShapeDtypeStruct((NP, 128), F32),
    )(al0, al1)


def _weights(L16, med, xls):
    BE = 2048

    def body(l_ref, m_ref, xl_ref, wc_ref, e_ref):
        lv = l_ref[...]
        L = lv[:, 0:8]
        live = lv[:, 8:9] > 0
        E = jnp.exp(jnp.minimum(L - m_ref[...][:, 0:8], 60.0))
        E = jnp.where(live, E, 0.0)
        e_ref[...] = jnp.concatenate([E, jnp.zeros((BE, 8), F32)], axis=1)
        xv = xl_ref[...]
        parts = []
        for ch in range(4):
            e2 = E[:, 2 * ch:2 * ch + 2]
            er = jnp.broadcast_to(e2[:, :, None], (BE, 2, HD)).reshape(BE, 128)
            parts.append((xv[:, ch * 128:(ch + 1) * 128] * er)[None])
        wc_ref[...] = jnp.concatenate(parts, axis=0)

    return pl.pallas_call(
        body,
        grid=(EP // BE,),
        in_specs=[
            pl.BlockSpec((BE, 16), lambda i: (i, 0)),
            pl.BlockSpec((BE, 128), lambda i: (i, 0)),
            pl.BlockSpec((BE, CH), lambda i: (i, 0)),
        ],
        out_specs=[
            pl.BlockSpec((4, BE, 128), lambda i: (0, i, 0)),
            pl.BlockSpec((BE, 16), lambda i: (i, 0)),
        ],
        out_shape=[
            jax.ShapeDtypeStruct((4, EP, 128), F32),
            jax.ShapeDtypeStruct((EP, 16), F32),
        ],
    )(L16, med, xls)


@functools.lru_cache(None)
def _make_finalize(residual):
    BM = 2048

    def body(*refs):
        ws = refs[0:8]
        e0_ref, e1_ref, b_ref = refs[8:11]
        if residual:
            hp_ref, h_ref, hc_ref = refs[11:14]
        else:
            h_ref, hc_ref = refs[11:13]
        s = (e0_ref[...] + e1_ref[...])[:, 0:HEADS]
        parts = []
        for ch in range(4):
            a = ws[2 * ch][...] + ws[2 * ch + 1][...]
            s2 = s[:, 2 * ch:2 * ch + 2]
            den = jnp.broadcast_to(s2[:, :, None], (BM, 2, HD)).reshape(BM, 128)
            den = jnp.where(den > 0, den, 1.0)
            parts.append(a / den + b_ref[...][:, ch * 128:(ch + 1) * 128])
        h = jax.nn.relu(jnp.concatenate(parts, axis=1))
        if residual:
            h = h + hp_ref[...]
        row = pl.program_id(0) * BM + lax.broadcasted_iota(I32, (BM, 1), 0)
        h = jnp.where(row < NN, h, 0.0)
        h_ref[...] = h
        hc_ref[...] = jnp.concatenate(
            [h[:, ch * 128:(ch + 1) * 128][None] for ch in range(4)], axis=0)

    in_specs = ([pl.BlockSpec((BM, 128), lambda i: (i, 0))] * 8
                + [pl.BlockSpec((BM, 16), lambda i: (i, 0))] * 2
                + [pl.BlockSpec((1, CH), lambda i: (0, 0))])
    if residual:
        in_specs.append(pl.BlockSpec((BM, CH), lambda i: (i, 0)))

    return pl.pallas_call(
        body,
        grid=(NP // BM,),
        in_specs=in_specs,
        out_specs=[
            pl.BlockSpec((BM, CH), lambda i: (i, 0)),
            pl.BlockSpec((4, BM, 128), lambda i: (0, i, 0)),
        ],
        out_shape=[
            jax.ShapeDtypeStruct((NP, CH), F32),
            jax.ShapeDtypeStruct((4, NP, 128), F32),
        ],
    )


def _head(abs_, cnts, p2t, pb, v1t, v1b, v2t, v2b):
    def body(a0, a1, a2, a3, a4, a5, a6, a7, c0, c1, p2_ref, pb_ref,
             v1_ref, vb1_ref, v2_ref, vb2_ref, pi_ref, v_ref):
        cnt = jnp.maximum((c0[...] + c1[...])[:, 0:1], 1.0)
        aa = (a0, a1, a2, a3, a4, a5, a6, a7)
        parts = [(aa[2 * ch][...] + aa[2 * ch + 1][...]) / cnt for ch in range(4)]
        s = jnp.concatenate(parts, axis=1)
        pi = jnp.dot(s, p2_ref[...], preferred_element_type=F32) + pb_ref[...]
        m = jnp.max(pi, axis=1, keepdims=True)
        lse = jnp.log(jnp.sum(jnp.exp(pi - m), axis=1, keepdims=True)) + m
        pi_ref[...] = pi - lse
        v = jax.nn.relu(jnp.dot(s, v1_ref[...], preferred_element_type=F32) + vb1_ref[...])
        v_ref[...] = jnp.dot(v, v2_ref[...], preferred_element_type=F32) + vb2_ref[...]

    return pl.pallas_call(
        body,
        out_shape=[
            jax.ShapeDtypeStruct((NG, CH), F32),
            jax.ShapeDtypeStruct((NG, 128), F32),
        ],
    )(*abs_, *cnts, p2t, pb, v1t, v1b, v2t, v2b)


def _pad_idx(a, n):
    return jnp.pad(a.astype(I32), (0, n - a.shape[0])).reshape(-1, 128)


def kernel(x, edge_index, edge_attr, batch, params):
    p = params
    emb = jnp.pad(p['state_number_embed'], ((0, 0), (0, 96)))
    remb = jnp.pad(p['regex_embed'], ((0, 0), (0, 96)))

    states2 = _pad_idx(x[:, 0], NP)
    src2 = _pad_idx(edge_index[0], EP)
    dst2 = _pad_idx(edge_index[1], EP)
    src_sn2 = _pad_idx(edge_attr[:, -2], EP)
    tgt_sn2 = _pad_idx(edge_attr[:, -1], EP)
    regex2 = _pad_idx(edge_attr[:, :8].reshape(-1), EP * 8)
    batch2 = _pad_idx(batch, NP)
    addl_pad = jnp.pad(x[:, 1:].astype(F32), ((0, NP - NN), (0, 126)))
    ones16 = jnp.pad((jnp.arange(NP) < NN).astype(F32)[:, None], ((0, 0), (0, 15)))
    zeros_n128 = jnp.zeros((NP, 128), F32)
    zeros_n16 = jnp.zeros((NP, 16), F32)
    zeros_g128 = jnp.zeros((NG, 128), F32)
    zeros_g16 = jnp.zeros((NG, 16), F32)

    wih_t = p['lstm_W_ih'].T
    whh_t = p['lstm_W_hh'].T
    lb = (p['lstm_b_ih'] + p['lstm_b_hh'])[None]

    emb_s = _gather(emb, states2)
    src_e = _gather(emb, src_sn2)
    tgt_e = _gather(emb, tgt_sn2)
    xg = _gather(remb, regex2).reshape(EP, 8, 128)

    enc, pv_src, pv_dst = _lstm(xg, tgt_e, src_e, wih_t, whh_t, lb)

    acc_s = _scatter(pv_src[None], src2, NP, zeros_n128)
    acc_d = _scatter(pv_dst[None], dst2, NP, zeros_n128)

    h = _hbuild(emb_s, addl_pad, acc_s[0, 0], acc_s[0, 1],
                acc_d[0, 0], acc_d[0, 1])

    hc = None
    for li, name in enumerate(['conv1', 'conv2', 'conv3', 'conv4']):
        cp = p[name]
        K = h.shape[1]
        wl = jnp.pad(cp['Wl'], ((0, K - cp['Wl'].shape[0]), (0, 0)))
        wr = jnp.pad(cp['Wr'], ((0, K - cp['Wr'].shape[0]), (0, 0)))
        xl, xr = _xlxr(h, wl, cp['bl'][None], wr, cp['br'][None])
        xls = _gather(xl, src2)
        xrd = _gather(xr, dst2)
        attf = cp['att'].reshape(1, CH)
        L16 = _ee_logits(xls, xrd, enc, cp['We'], attf)
        accl = _scatter(L16[None], dst2, NP, zeros_n16)
        med = _segmean(accl[0, 0], accl[0, 1])
        mede = _gather(med, dst2)
        wc, e16 = _weights(L16, mede, xls)
        acc_w = _scatter(wc, dst2, NP, zeros_n128)
        acc_e = _scatter(e16[None], dst2, NP, zeros_n16)
        fin = _make_finalize(li > 0)
        args = tuple(acc_w[ch, c] for ch in range(4) for c in range(2)) + (
            acc_e[0, 0], acc_e[0, 1], cp['bias'][None])
        if li > 0:
            args = args + (h,)
        h, hc = fin(*args)

    acc_b = _scatter(hc, batch2, NG, zeros_g128)
    cnt_b = _scatter(ones16[None], batch2, NG, zeros_g16)

    p2t = p['policy_head2_W'].T
    v1t = p['value_head1_W'].T
    v2t = jnp.pad(p['value_head2_W'].T, ((0, 0), (0, 127)))
    v2b = jnp.pad(p['value_head2_b'][None], ((0, 0), (0, 127)))
    abs_ = tuple(acc_b[ch, c] for ch in range(4) for c in range(2))
    cnts = (cnt_b[0, 0], cnt_b[0, 1])
    logpi, vout = _head(abs_, cnts, p2t, p['policy_head2_b'][None],
                        v1t, p['value_head1_b'][None], v2t, v2b)
    return logpi, vout[:, :1]


# speedup vs baseline: 6.2172x; 1.0042x over previous
import functools

import jax
import jax.numpy as jnp
from jax import lax
from jax.experimental import pallas as pl
from jax.experimental.pallas import tpu as pltpu
from jax.experimental.pallas import tpu_sc as plsc

F32 = jnp.float32
I32 = jnp.int32

NN = 10000
NE = 160000
NP = 10240
EP = 163840
NG = 64
HEADS = 8
CH = 512
HD = 64
LD = 64
NW = 32


def _sc_mesh():
    return plsc.VectorSubcoreMesh(core_axis_name="c", subcore_axis_name="s",
                                  num_cores=2, num_subcores=16)


@functools.lru_cache(None)
def _make_gather(T, D, NSB):
    nper = -(-NSB // NW)
    paired = NSB % (2 * NW) == 0 and D <= 128

    if paired:
        @functools.partial(
            pl.kernel,
            out_type=jax.ShapeDtypeStruct((NSB * 128, D), F32),
            mesh=_sc_mesh(),
            scratch_types=[
                pltpu.VMEM((128,), I32),
                pltpu.VMEM((128,), I32),
                pltpu.VMEM((128, D), F32),
                pltpu.VMEM((128, D), F32),
                pltpu.SemaphoreType.DMA,
                pltpu.SemaphoreType.DMA,
            ],
        )
        def k(table_hbm, idx_hbm, out_hbm, idx_a, idx_b, rows_a, rows_b,
              sem_a, sem_b):
            wid = lax.axis_index("s") * 2 + lax.axis_index("c")

            def body(i, carry):
                sb1 = wid + (2 * i) * NW
                sb2 = wid + (2 * i + 1) * NW
                pltpu.sync_copy(idx_hbm.at[sb1], idx_a)
                pltpu.sync_copy(idx_hbm.at[sb2], idx_b)
                h1 = pltpu.async_copy(table_hbm.at[idx_a], rows_a, sem_a)
                h2 = pltpu.async_copy(table_hbm.at[idx_b], rows_b, sem_b)
                h1.wait()
                pltpu.sync_copy(rows_a, out_hbm.at[pl.ds(sb1 * 128, 128)])
                h2.wait()
                pltpu.sync_copy(rows_b, out_hbm.at[pl.ds(sb2 * 128, 128)])
                return carry

            lax.fori_loop(0, nper // 2, body, 0)

        return k

    @functools.partial(
        pl.kernel,
        out_type=jax.ShapeDtypeStruct((NSB * 128, D), F32),
        mesh=_sc_mesh(),
        scratch_types=[
            pltpu.VMEM((128,), I32),
            pltpu.VMEM((128, D), F32),
            pltpu.SemaphoreType.DMA,
        ],
    )
    def k(table_hbm, idx_hbm, out_hbm, idx_v, rows_v, sem):
        wid = lax.axis_index("s") * 2 + lax.axis_index("c")

        def body(i, carry):
            sb = wid + i * NW

            @pl.when(sb < NSB)
            def _():
                pltpu.sync_copy(idx_hbm.at[sb], idx_v)
                pltpu.async_copy(table_hbm.at[idx_v], rows_v, sem).wait()
                pltpu.sync_copy(rows_v, out_hbm.at[pl.ds(sb * 128, 128)])

            return carry

        lax.fori_loop(0, nper, body, 0)

    return k


def _gather(table, idx2):
    T, D = table.shape
    NSB = idx2.shape[0]
    return _make_gather(T, D, NSB)(table, idx2)


@functools.lru_cache(None)
def _make_scatter(N, Dc, NCH, NSB):
    nper = -(-NSB // NW)
    ZT = min(16, N // 8)
    rpt = N // ZT

    pipelined = NSB % NW == 0

    @functools.partial(
        pl.kernel,
        out_type=jax.ShapeDtypeStruct((NCH, 2, N, Dc), F32),
        mesh=_sc_mesh(),
        scratch_types=[
            pltpu.VMEM((128,), I32),
            pltpu.VMEM((128, Dc), F32),
            pltpu.VMEM((128, Dc), F32),
            pltpu.VMEM_SHARED((N, Dc), F32),
            pltpu.SemaphoreType.DMA,
            pltpu.SemaphoreType.DMA,
        ],
    )
    def k(vals_hbm, idx_hbm, zeros_hbm, out_hbm, idx_v, val_a, val_b, acc,
          sem_a, sem_b):
        cc = lax.axis_index("c")
        ss = lax.axis_index("s")
        wid = ss * 2 + cc
        bufs = [(val_a, sem_a), (val_b, sem_b)]
        for ch in range(NCH):
            @pl.when(ss < ZT)
            def _():
                pltpu.sync_copy(zeros_hbm.at[pl.ds(ss * rpt, rpt)],
                                acc.at[pl.ds(ss * rpt, rpt)])
            plsc.subcore_barrier()

            if pipelined:
                buf0, s0 = bufs[0]
                cur = pltpu.async_copy(
                    vals_hbm.at[ch, pl.ds(wid * 128, 128)], buf0, s0)
                for i in range(nper):
                    nxt = None
                    if i + 1 < nper:
                        bufn, sn = bufs[(i + 1) % 2]
                        nxt = pltpu.async_copy(
                            vals_hbm.at[ch, pl.ds((wid + (i + 1) * NW) * 128, 128)],
                            bufn, sn)
                    cur.wait()
                    pltpu.sync_copy(idx_hbm.at[wid + i * NW], idx_v)
                    pltpu.sync_copy(bufs[i % 2][0], acc.at[idx_v], add=True)
                    cur = nxt
            else:
                def body(i, carry):
                    sb = wid + i * NW

                    @pl.when(sb < NSB)
                    def _():
                        pltpu.sync_copy(idx_hbm.at[sb], idx_v)
                        pltpu.sync_copy(vals_hbm.at[ch, pl.ds(sb * 128, 128)],
                                        val_a)
                        pltpu.sync_copy(val_a, acc.at[idx_v], add=True)

                    return carry

                lax.fori_loop(0, nper, body, 0)
            plsc.subcore_barrier()

            @pl.when(ss < ZT)
            def _():
                pltpu.sync_copy(acc.at[pl.ds(ss * rpt, rpt)],
                                out_hbm.at[ch, cc, pl.ds(ss * rpt, rpt)])

    return k


def _scatter(values, idx2, N, zeros):
    NCH, ROWS, Dc = values.shape
    NSB = idx2.shape[0]
    return _make_scatter(N, Dc, NCH, NSB)(values, idx2, zeros)


def _xlxr(X, Wl, bl, Wr, br):
    M, K = X.shape
    N = Wl.shape[1]
    BM = 1024

    def body(x_ref, wl_ref, bl_ref, wr_ref, br_ref, o1_ref, o2_ref):
        xv = x_ref[...]
        o1_ref[...] = jnp.dot(xv, wl_ref[...], preferred_element_type=F32) + bl_ref[...]
        o2_ref[...] = jnp.dot(xv, wr_ref[...], preferred_element_type=F32) + br_ref[...]

    return pl.pallas_call(
        body,
        grid=(M // BM,),
        in_specs=[
            pl.BlockSpec((BM, K), lambda i: (i, 0)),
            pl.BlockSpec((K, N), lambda i: (0, 0)),
            pl.BlockSpec((1, N), lambda i: (0, 0)),
            pl.BlockSpec((K, N), lambda i: (0, 0)),
            pl.BlockSpec((1, N), lambda i: (0, 0)),
        ],
        out_specs=[
            pl.BlockSpec((BM, N), lambda i: (i, 0)),
            pl.BlockSpec((BM, N), lambda i: (i, 0)),
        ],
        out_shape=[
            jax.ShapeDtypeStruct((M, N), F32),
            jax.ShapeDtypeStruct((M, N), F32),
        ],
    )(X, Wl, bl, Wr, br)


def _lstm(xg, tgt_e, src_e, wih_t, whh_t, b):
    BE = 2048

    def body(xg_ref, te_ref, se_ref, wih_ref, whh_ref, b_ref, enc_ref, ps_ref, pd_ref):
        xv = xg_ref[...][:, :, 0:32]
        wih = wih_ref[...]
        whh = whh_ref[...]
        bb = b_ref[...]
        h = jnp.zeros((BE, LD), F32)
        c = jnp.zeros((BE, LD), F32)
        for t in range(8):
            g = (jnp.dot(xv[:, t, :], wih, preferred_element_type=F32)
                 + jnp.dot(h, whh, preferred_element_type=F32) + bb)
            gi = jax.nn.sigmoid(g[:, 0:LD])
            gf = jax.nn.sigmoid(g[:, LD:2 * LD])
            gg = jnp.tanh(g[:, 2 * LD:3 * LD])
            go = jax.nn.sigmoid(g[:, 3 * LD:4 * LD])
            c = gf * c + gi * gg
            h = go * jnp.tanh(c)
        row = pl.program_id(0) * BE + lax.broadcasted_iota(I32, (BE, 1), 0)
        mask = row < NE
        enc_ref[...] = jnp.where(mask, h, 0.0)
        ones = jnp.ones((BE, 1), F32)
        zpad = jnp.zeros((BE, 31), F32)
        pvs = jnp.concatenate([te_ref[...][:, 0:32], h, ones, zpad], axis=1)
        pvd = jnp.concatenate([se_ref[...][:, 0:32], h, ones, zpad], axis=1)
        ps_ref[...] = jnp.where(mask, pvs, 0.0)
        pd_ref[...] = jnp.where(mask, pvd, 0.0)

    return pl.pallas_call(
        body,
        grid=(EP // BE,),
        in_specs=[
            pl.BlockSpec((BE, 8, 128), lambda i: (i, 0, 0)),
            pl.BlockSpec((BE, 128), lambda i: (i, 0)),
            pl.BlockSpec((BE, 128), lambda i: (i, 0)),
            pl.BlockSpec((32, 256), lambda i: (0, 0)),
            pl.BlockSpec((64, 256), lambda i: (0, 0)),
            pl.BlockSpec((1, 256), lambda i: (0, 0)),
        ],
        out_specs=[
            pl.BlockSpec((BE, LD), lambda i: (i, 0)),
            pl.BlockSpec((BE, 128), lambda i: (i, 0)),
            pl.BlockSpec((BE, 128), lambda i: (i, 0)),
        ],
        out_shape=[
            jax.ShapeDtypeStruct((EP, LD), F32),
            jax.ShapeDtypeStruct((EP, 128), F32),
            jax.ShapeDtypeStruct((EP, 128), F32),
        ],
    )(xg, tgt_e, src_e, wih_t, whh_t, b)


def _hbuild(emb_s, addl, as0, as1, ad0, ad1):
    BM = 2048

    def body(es_ref, adl_ref, as0_ref, as1_ref, ad0_ref, ad1_ref, o_ref):
        es = es_ref[...][:, 0:32]
        a_s = as0_ref[...] + as1_ref[...]
        a_d = ad0_ref[...] + ad1_ref[...]
        cs = jnp.maximum(a_s[:, 96:97], 1.0)
        cd = jnp.maximum(a_d[:, 96:97], 1.0)
        out_trans = a_s[:, 0:96] / cs
        in_trans = a_d[:, 0:96] / cd
        zpad = jnp.zeros((BM, 30), F32)
        o_ref[...] = jnp.concatenate(
            [es, adl_ref[...][:, 0:2], in_trans, out_trans, zpad], axis=1)

    return pl.pallas_call(
        body,
        grid=(NP // BM,),
        in_specs=[pl.BlockSpec((BM, 128), lambda i: (i, 0))] * 6,
        out_specs=pl.BlockSpec((BM, 256), lambda i: (i, 0)),
        out_shape=jax.ShapeDtypeStruct((NP, 256), F32),
    )(emb_s, addl, as0, as1, ad0, ad1)


def _ee_logits(xls, xrd, enc, we, attf):
    BE = 2048

    def body(xl_ref, xr_ref, en_ref, we_ref, at_ref, l_ref):
        ee = jnp.dot(en_ref[...], we_ref[...], preferred_element_type=F32)
        t = xl_ref[...] + xr_ref[...] + ee
        t = jnp.where(t > 0, t, 0.2 * t)
        tw = t * at_ref[...]
        L = jnp.sum(tw.reshape(BE, HEADS, HD), axis=2)
        row = pl.program_id(0) * BE + lax.broadcasted_iota(I32, (BE, 1), 0)
        mask = row < NE
        one = jnp.where(mask, jnp.ones((BE, 1), F32), 0.0)
        Lm = jnp.where(mask, L, 0.0)
        l_ref[...] = jnp.concatenate([Lm, one, jnp.zeros((BE, 7), F32)], axis=1)

    return pl.pallas_call(
        body,
        grid=(EP // BE,),
        in_specs=[
            pl.BlockSpec((BE, CH), lambda i: (i, 0)),
            pl.BlockSpec((BE, CH), lambda i: (i, 0)),
            pl.BlockSpec((BE, LD), lambda i: (i, 0)),
            pl.BlockSpec((LD, CH), lambda i: (0, 0)),
            pl.BlockSpec((1, CH), lambda i: (0, 0)),
        ],
        out_specs=pl.BlockSpec((BE, 16), lambda i: (i, 0)),
        out_shape=jax.ShapeDtypeStruct((EP, 16), F32),
    )(xls, xrd, enc, we, attf)


def _segmean(al0, al1):
    BM = 2048

    def body(a0_ref, a1_ref, o_ref):
        a = a0_ref[...] + a1_ref[...]
        cnt = jnp.maximum(a[:, 8:9], 1.0)
        o_ref[...] = jnp.concatenate(
            [a[:, 0:8] / cnt, jnp.zeros((BM, 120), F32)], axis=1)

    return pl.pallas_call(
        body,
        grid=(NP // BM,),
        in_specs=[pl.BlockSpec((BM, 16), lambda i: (i, 0))] * 2,
        out_specs=pl.BlockSpec((BM, 128), lambda i: (i, 0)),
        out_shape=jax.ShapeDtypeStruct((NP, 128), F32),
    )(al0, al1)


def _weights(L16, med, xls):
    BE = 2048

    def body(l_ref, m_ref, xl_ref, wc_ref, e_ref):
        lv = l_ref[...]
        L = lv[:, 0:8]
        live = lv[:, 8:9] > 0
        E = jnp.exp(jnp.minimum(L - m_ref[...][:, 0:8], 60.0))
        E = jnp.where(live, E, 0.0)
        e_ref[...] = jnp.concatenate([E, jnp.zeros((BE, 8), F32)], axis=1)
        xv = xl_ref[...]
        parts = []
        for ch in range(4):
            e2 = E[:, 2 * ch:2 * ch + 2]
            er = jnp.broadcast_to(e2[:, :, None], (BE, 2, HD)).reshape(BE, 128)
            parts.append((xv[:, ch * 128:(ch + 1) * 128] * er)[None])
        wc_ref[...] = jnp.concatenate(parts, axis=0)

    return pl.pallas_call(
        body,
        grid=(EP // BE,),
        in_specs=[
            pl.BlockSpec((BE, 16), lambda i: (i, 0)),
            pl.BlockSpec((BE, 128), lambda i: (i, 0)),
            pl.BlockSpec((BE, CH), lambda i: (i, 0)),
        ],
        out_specs=[
            pl.BlockSpec((4, BE, 128), lambda i: (0, i, 0)),
            pl.BlockSpec((BE, 16), lambda i: (i, 0)),
        ],
        out_shape=[
            jax.ShapeDtypeStruct((4, EP, 128), F32),
            jax.ShapeDtypeStruct((EP, 16), F32),
        ],
    )(L16, med, xls)


@functools.lru_cache(None)
def _make_finalize(residual):
    BM = 2048

    def body(*refs):
        ws = refs[0:8]
        e0_ref, e1_ref, b_ref = refs[8:11]
        if residual:
            hp_ref, h_ref, hc_ref = refs[11:14]
        else:
            h_ref, hc_ref = refs[11:13]
        s = (e0_ref[...] + e1_ref[...])[:, 0:HEADS]
        parts = []
        for ch in range(4):
            a = ws[2 * ch][...] + ws[2 * ch + 1][...]
            s2 = s[:, 2 * ch:2 * ch + 2]
            den = jnp.broadcast_to(s2[:, :, None], (BM, 2, HD)).reshape(BM, 128)
            den = jnp.where(den > 0, den, 1.0)
            parts.append(a / den + b_ref[...][:, ch * 128:(ch + 1) * 128])
        h = jax.nn.relu(jnp.concatenate(parts, axis=1))
        if residual:
            h = h + hp_ref[...]
        row = pl.program_id(0) * BM + lax.broadcasted_iota(I32, (BM, 1), 0)
        h = jnp.where(row < NN, h, 0.0)
        h_ref[...] = h
        hc_ref[...] = jnp.concatenate(
            [h[:, ch * 128:(ch + 1) * 128][None] for ch in range(4)], axis=0)

    in_specs = ([pl.BlockSpec((BM, 128), lambda i: (i, 0))] * 8
                + [pl.BlockSpec((BM, 16), lambda i: (i, 0))] * 2
                + [pl.BlockSpec((1, CH), lambda i: (0, 0))])
    if residual:
        in_specs.append(pl.BlockSpec((BM, CH), lambda i: (i, 0)))

    return pl.pallas_call(
        body,
        grid=(NP // BM,),
        in_specs=in_specs,
        out_specs=[
            pl.BlockSpec((BM, CH), lambda i: (i, 0)),
            pl.BlockSpec((4, BM, 128), lambda i: (0, i, 0)),
        ],
        out_shape=[
            jax.ShapeDtypeStruct((NP, CH), F32),
            jax.ShapeDtypeStruct((4, NP, 128), F32),
        ],
    )


def _head(abs_, cnts, p2t, pb, v1t, v1b, v2t, v2b):
    def body(a0, a1, a2, a3, a4, a5, a6, a7, c0, c1, p2_ref, pb_ref,
             v1_ref, vb1_ref, v2_ref, vb2_ref, pi_ref, v_ref):
        cnt = jnp.maximum((c0[...] + c1[...])[:, 0:1], 1.0)
        aa = (a0, a1, a2, a3, a4, a5, a6, a7)
        parts = [(aa[2 * ch][...] + aa[2 * ch + 1][...]) / cnt for ch in range(4)]
        s = jnp.concatenate(parts, axis=1)
        pi = jnp.dot(s, p2_ref[...], preferred_element_type=F32) + pb_ref[...]
        m = jnp.max(pi, axis=1, keepdims=True)
        lse = jnp.log(jnp.sum(jnp.exp(pi - m), axis=1, keepdims=True)) + m
        pi_ref[...] = pi - lse
        v = jax.nn.relu(jnp.dot(s, v1_ref[...], preferred_element_type=F32) + vb1_ref[...])
        v_ref[...] = jnp.dot(v, v2_ref[...], preferred_element_type=F32) + vb2_ref[...]

    return pl.pallas_call(
        body,
        out_shape=[
            jax.ShapeDtypeStruct((NG, CH), F32),
            jax.ShapeDtypeStruct((NG, 128), F32),
        ],
    )(*abs_, *cnts, p2t, pb, v1t, v1b, v2t, v2b)


def _pad_idx(a, n):
    return jnp.pad(a.astype(I32), (0, n - a.shape[0])).reshape(-1, 128)


def kernel(x, edge_index, edge_attr, batch, params):
    p = params
    emb = jnp.pad(p['state_number_embed'], ((0, 0), (0, 96)))
    remb = jnp.pad(p['regex_embed'], ((0, 0), (0, 96)))

    states2 = _pad_idx(x[:, 0], NP)
    src2 = _pad_idx(edge_index[0], EP)
    dst2 = _pad_idx(edge_index[1], EP)
    src_sn2 = _pad_idx(edge_attr[:, -2], EP)
    tgt_sn2 = _pad_idx(edge_attr[:, -1], EP)
    regex2 = _pad_idx(edge_attr[:, :8].reshape(-1), EP * 8)
    batch2 = _pad_idx(batch, NP)
    addl_pad = jnp.pad(x[:, 1:].astype(F32), ((0, NP - NN), (0, 126)))
    ones16 = jnp.pad((jnp.arange(NP) < NN).astype(F32)[:, None], ((0, 0), (0, 15)))
    zeros_n128 = jnp.zeros((NP, 128), F32)
    zeros_n16 = jnp.zeros((NP, 16), F32)
    zeros_g128 = jnp.zeros((NG, 128), F32)
    zeros_g16 = jnp.zeros((NG, 16), F32)

    wih_t = p['lstm_W_ih'].T
    whh_t = p['lstm_W_hh'].T
    lb = (p['lstm_b_ih'] + p['lstm_b_hh'])[None]

    emb_s = _gather(emb, states2)
    src_e = _gather(emb, src_sn2)
    tgt_e = _gather(emb, tgt_sn2)
    xg = _gather(remb, regex2).reshape(EP, 8, 128)

    enc, pv_src, pv_dst = _lstm(xg, tgt_e, src_e, wih_t, whh_t, lb)

    acc_s = _scatter(pv_src[None], src2, NP, zeros_n128)
    acc_d = _scatter(pv_dst[None], dst2, NP, zeros_n128)

    h = _hbuild(emb_s, addl_pad, acc_s[0, 0], acc_s[0, 1],
                acc_d[0, 0], acc_d[0, 1])

    hc = None
    for li, name in enumerate(['conv1', 'conv2', 'conv3', 'conv4']):
        cp = p[name]
        K = h.shape[1]
        wl = jnp.pad(cp['Wl'], ((0, K - cp['Wl'].shape[0]), (0, 0)))
        wr = jnp.pad(cp['Wr'], ((0, K - cp['Wr'].shape[0]), (0, 0)))
        xl, xr = _xlxr(h, wl, cp['bl'][None], wr, cp['br'][None])
        xls = _gather(xl, src2)
        xrd = _gather(xr, dst2)
        attf = cp['att'].reshape(1, CH)
        L16 = _ee_logits(xls, xrd, enc, cp['We'], attf)
        accl = _scatter(L16[None], dst2, NP, zeros_n16)
        med = _segmean(accl[0, 0], accl[0, 1])
        mede = _gather(med, dst2)
        wc, e16 = _weights(L16, mede, xls)
        acc_w = _scatter(wc, dst2, NP, zeros_n128)
        acc_e = _scatter(e16[None], dst2, NP, zeros_n16)
        fin = _make_finalize(li > 0)
        args = tuple(acc_w[ch, c] for ch in range(4) for c in range(2)) + (
            acc_e[0, 0], acc_e[0, 1], cp['bias'][None])
        if li > 0:
            args = args + (h,)
        h, hc = fin(*args)

    acc_b = _scatter(hc, batch2, NG, zeros_g128)
    cnt_b = _scatter(ones16[None], batch2, NG, zeros_g16)

    p2t = p['policy_head2_W'].T
    v1t = p['value_head1_W'].T
    v2t = jnp.pad(p['value_head2_W'].T, ((0, 0), (0, 127)))
    v2b = jnp.pad(p['value_head2_b'][None], ((0, 0), (0, 127)))
    abs_ = tuple(acc_b[ch, c] for ch in range(4) for c in range(2))
    cnts = (cnt_b[0, 0], cnt_b[0, 1])
    logpi, vout = _head(abs_, cnts, p2t, p['policy_head2_b'][None],
                        v1t, p['value_head1_b'][None], v2t, v2b)
    return logpi, vout[:, :1]
